# baseline (speedup 1.0000x reference)
import functools

import jax
import jax.numpy as jnp
from jax import lax
from jax.experimental import pallas as pl
from jax.experimental.pallas import tpu as pltpu
from jax.experimental.pallas import tpu_sc as plsc

D = 64
L = 16
IDXW = 128
CHUNK = 256
NW = 32
OW = 128


def _lookup(idx2d, w128):
    nidx_rows = idx2d.shape[0]
    b_total = nidx_rows * IDXW
    per_w = b_total // NW
    nch = per_w // CHUNK
    g_per_chunk = CHUNK // IDXW
    idx_rows_per_w = per_w // IDXW
    orows_chunk = CHUNK * (D // 2) // OW
    orows_w = per_w * (D // 2) // OW
    orows_total = b_total * (D // 2) // OW

    mesh = plsc.VectorSubcoreMesh(core_axis_name="c", subcore_axis_name="s")

    @functools.partial(
        pl.kernel,
        out_type=jax.ShapeDtypeStruct((orows_total, OW), jnp.int32),
        mesh=mesh,
        scratch_types=[
            pltpu.VMEM((g_per_chunk, IDXW), jnp.int32),
            pltpu.VMEM((g_per_chunk, IDXW), jnp.int32),
            pltpu.VMEM((CHUNK,), jnp.int32),
            pltpu.VMEM((CHUNK, 2 * D), jnp.float32),
            pltpu.VMEM((orows_chunk, OW), jnp.int32),
            pltpu.SemaphoreType.DMA,
        ],
        compiler_params=pltpu.CompilerParams(
            needs_layout_passes=False, use_tc_tiling_on_sc=False
        ),
    )
    def run(idx_hbm, tbl_hbm, out_hbm, idx_v, phys_v, colb_v, rows_v,
            out_v, sem):
        cid = lax.axis_index("c")
        sid = lax.axis_index("s")
        wid = sid * 2 + cid
        idx_row0 = wid * idx_rows_per_w
        orow0 = wid * orows_w

        iota = lax.iota(jnp.int32, L)

        def chunk_body(t, carry):
            pltpu.sync_copy(
                idx_hbm.at[pl.ds(idx_row0 + t * g_per_chunk, g_per_chunk)],
                idx_v,
            )
            for g in range(g_per_chunk):
                for k in range(IDXW // L):
                    v = idx_v[g, pl.ds(k * L, L)]
                    phys_v[g, pl.ds(k * L, L)] = v >> 1
                    colb_v[pl.ds(g * IDXW + k * L, L)] = (v & 1) << 6
            cps = []
            for g in range(g_per_chunk):
                cps.append(
                    pltpu.async_copy(
                        tbl_hbm.at[phys_v.at[g]],
                        rows_v.at[pl.ds(g * IDXW, IDXW)],
                        sem,
                    )
                )
            for cp in cps:
                cp.wait()

            def cast_block(jb, c2):
                j0 = jb * L
                cbv = colb_v[pl.ds(j0, L)]
                for k in range(L):
                    j = j0 + k
                    cb = cbv[k]
                    jv = jnp.full((L,), j, jnp.int32)
                    for h in range(D // 32):
                        ev = plsc.load_gather(
                            rows_v, [jv, cb + h * 32 + 2 * iota]
                        )
                        od = plsc.load_gather(
                            rows_v, [jv, cb + h * 32 + 2 * iota + 1]
                        )
                        p = plsc.pack(
                            ev, od, format=plsc.PackFormat.INTERLEAVED
                        )
                        w = plsc.bitcast(p, jnp.int32)
                        wo = k * (D // 2) + h * L
                        out_v[jb * (L * (D // 2) // OW) + wo // OW,
                              pl.ds(wo % OW, L)] = w
                return c2

            lax.fori_loop(0, CHUNK // L, cast_block, 0)
            pltpu.sync_copy(
                out_v, out_hbm.at[pl.ds(orow0 + t * orows_chunk, orows_chunk)]
            )
            return carry

        lax.fori_loop(0, nch, chunk_body, 0)

    return run(idx2d, w128)


def kernel(input_ids, weight):
    b, s = input_ids.shape
    ids = input_ids.reshape(-1).astype(jnp.int32).reshape(-1, IDXW)
    w128 = weight.reshape(-1, 2 * D)
    owords = _lookup(ids, w128)
    out = jax.lax.bitcast_convert_type(owords, jnp.bfloat16)
    return out.reshape(b, s, D)


# speedup vs baseline: 10.3927x; 10.3927x over previous
import functools

import jax
import jax.numpy as jnp
from jax import lax
from jax.experimental import pallas as pl
from jax.experimental.pallas import tpu as pltpu
from jax.experimental.pallas import tpu_sc as plsc

D = 64
L = 16
CHUNK = 512
NG = 4
GROWS = CHUNK // NG
NW = 32
OWORDS = CHUNK * (D // 2)
UNROLL = 8


def _lookup(ids_flat, weight):
    b_total = ids_flat.shape[0]
    per_w = b_total // NW
    nch = per_w // CHUNK
    assert nch % 2 == 0

    mesh = plsc.VectorSubcoreMesh(core_axis_name="c", subcore_axis_name="s")

    @functools.partial(
        pl.kernel,
        out_type=jax.ShapeDtypeStruct((b_total * (D // 2),), jnp.int32),
        mesh=mesh,
        scratch_types=[
            pltpu.VMEM((CHUNK,), jnp.int32),
            pltpu.VMEM((CHUNK,), jnp.int32),
            pltpu.VMEM((CHUNK, D), jnp.float32),
            pltpu.VMEM((CHUNK, D), jnp.float32),
            pltpu.VMEM((OWORDS,), jnp.int32),
            pltpu.VMEM((OWORDS,), jnp.int32),
            pltpu.SemaphoreType.DMA,
            pltpu.SemaphoreType.DMA,
            pltpu.SemaphoreType.DMA,
            pltpu.SemaphoreType.DMA,
        ],
        compiler_params=pltpu.CompilerParams(
            needs_layout_passes=False, use_tc_tiling_on_sc=False
        ),
    )
    def run(idx_hbm, tbl_hbm, out_hbm, idx_a, idx_b, rows_a, rows_b,
            out_a, out_b, gsem_a, gsem_b, osem_a, osem_b):
        cid = lax.axis_index("c")
        sid = lax.axis_index("s")
        wid = sid * 2 + cid
        flat0 = wid * per_w
        oflat0 = wid * per_w * (D // 2)

        iota = lax.iota(jnp.int32, L)
        bufs = ((idx_a, rows_a, out_a, gsem_a, osem_a),
                (idx_b, rows_b, out_b, gsem_b, osem_b))

        def start(t, bi):
            idx_v, rows_v, _, gsem, _ = bufs[bi]
            pltpu.sync_copy(
                idx_hbm.at[pl.ds(flat0 + t * CHUNK, CHUNK)], idx_v
            )
            for g in range(NG):
                pltpu.async_copy(
                    tbl_hbm.at[idx_v.at[pl.ds(g * GROWS, GROWS)]],
                    rows_v.at[pl.ds(g * GROWS, GROWS)],
                    gsem,
                )

        def wait_gathers(t, bi):
            idx_v, rows_v, _, gsem, _ = bufs[bi]
            for g in range(NG):
                pltpu.make_async_copy(
                    tbl_hbm.at[idx_v.at[pl.ds(g * GROWS, GROWS)]],
                    rows_v.at[pl.ds(g * GROWS, GROWS)],
                    gsem,
                ).wait()

        def out_slice(t):
            return out_hbm.at[pl.ds(oflat0 + t * OWORDS, OWORDS)]

        def fire_out(t, bi):
            _, _, out_v, _, osem = bufs[bi]
            pltpu.async_copy(out_v, out_slice(t), osem)

        def wait_out(t, bi):
            _, _, out_v, _, osem = bufs[bi]
            pltpu.make_async_copy(out_v, out_slice(t), osem).wait()

        def cast(bi):
            _, rows_v, out_v, _, _ = bufs[bi]

            def cast_body(gi, c2):
                for u in range(UNROLL):
                    j = gi * (UNROLL // 2) + u // 2
                    jv = jnp.full((L,), j, jnp.int32)
                    c0 = (u % 2) * 32
                    ev = plsc.load_gather(rows_v, [jv, c0 + 2 * iota])
                    od = plsc.load_gather(rows_v, [jv, c0 + 2 * iota + 1])
                    p = plsc.pack(ev, od, format=plsc.PackFormat.INTERLEAVED)
                    w = plsc.bitcast(p, jnp.int32)
                    out_v[pl.ds(gi * (UNROLL * L) + u * L, L)] = w
                return c2

            lax.fori_loop(0, OWORDS // (UNROLL * L), cast_body, 0)

        start(0, 0)

        def body(p, carry):
            t0 = 2 * p
            t1 = 2 * p + 1
            start(t1, 1)
            wait_gathers(t0, 0)

            @pl.when(p > 0)
            def _():
                wait_out(t0 - 2, 0)

            cast(0)
            fire_out(t0, 0)

            @pl.when(p < nch // 2 - 1)
            def _():
                start(t0 + 2, 0)

            wait_gathers(t1, 1)

            @pl.when(p > 0)
            def _():
                wait_out(t1 - 2, 1)

            cast(1)
            fire_out(t1, 1)
            return carry

        lax.fori_loop(0, nch // 2, body, 0)
        wait_out(nch - 2, 0)
        wait_out(nch - 1, 1)

    return run(ids_flat, weight)


def kernel(input_ids, weight):
    b, s = input_ids.shape
    ids = input_ids.reshape(-1).astype(jnp.int32)
    owords = _lookup(ids, weight)
    out = jax.lax.bitcast_convert_type(owords, jnp.bfloat16)
    return out.reshape(b, s, D)


# speedup vs baseline: 11.9258x; 1.1475x over previous
import functools

import jax
import jax.numpy as jnp
from jax import lax
from jax.experimental import pallas as pl
from jax.experimental.pallas import tpu as pltpu
from jax.experimental.pallas import tpu_sc as plsc

D = 64
L = 16
CHUNK = 512
NG = 4
GROWS = CHUNK // NG
NW = 32
OWORDS = CHUNK * (D // 2)
UNROLL = 8


def _lookup(ids_flat, weight):
    b_total = ids_flat.shape[0]
    per_w = b_total // NW
    nch = per_w // CHUNK
    assert nch % 2 == 0

    mesh = plsc.VectorSubcoreMesh(core_axis_name="c", subcore_axis_name="s")

    @functools.partial(
        pl.kernel,
        out_type=jax.ShapeDtypeStruct((b_total * D,), jnp.bfloat16),
        mesh=mesh,
        scratch_types=[
            pltpu.VMEM((CHUNK,), jnp.int32),
            pltpu.VMEM((CHUNK,), jnp.int32),
            pltpu.VMEM((CHUNK, D), jnp.float32),
            pltpu.VMEM((CHUNK, D), jnp.float32),
            pltpu.VMEM((CHUNK * D,), jnp.bfloat16),
            pltpu.VMEM((CHUNK * D,), jnp.bfloat16),
            pltpu.SemaphoreType.DMA,
            pltpu.SemaphoreType.DMA,
            pltpu.SemaphoreType.DMA,
            pltpu.SemaphoreType.DMA,
        ],
        compiler_params=pltpu.CompilerParams(
            needs_layout_passes=False, use_tc_tiling_on_sc=False
        ),
    )
    def run(idx_hbm, tbl_hbm, out_hbm, idx_a, idx_b, rows_a, rows_b,
            out_a, out_b, gsem_a, gsem_b, osem_a, osem_b):
        cid = lax.axis_index("c")
        sid = lax.axis_index("s")
        wid = sid * 2 + cid
        flat0 = wid * per_w
        oflat0 = wid * per_w * D

        iota = lax.iota(jnp.int32, L)
        bufs = ((idx_a, rows_a, out_a, gsem_a, osem_a),
                (idx_b, rows_b, out_b, gsem_b, osem_b))

        def start(t, bi):
            idx_v, rows_v, _, gsem, _ = bufs[bi]
            pltpu.sync_copy(
                idx_hbm.at[pl.ds(flat0 + t * CHUNK, CHUNK)], idx_v
            )
            for g in range(NG):
                pltpu.async_copy(
                    tbl_hbm.at[idx_v.at[pl.ds(g * GROWS, GROWS)]],
                    rows_v.at[pl.ds(g * GROWS, GROWS)],
                    gsem,
                )

        def wait_gathers(t, bi):
            idx_v, rows_v, _, gsem, _ = bufs[bi]
            for g in range(NG):
                pltpu.make_async_copy(
                    tbl_hbm.at[idx_v.at[pl.ds(g * GROWS, GROWS)]],
                    rows_v.at[pl.ds(g * GROWS, GROWS)],
                    gsem,
                ).wait()

        def out_slice(t):
            return out_hbm.at[pl.ds(oflat0 + t * CHUNK * D, CHUNK * D)]

        def fire_out(t, bi):
            _, _, out_v, _, osem = bufs[bi]
            pltpu.async_copy(out_v, out_slice(t), osem)

        def wait_out(t, bi):
            _, _, out_v, _, osem = bufs[bi]
            pltpu.make_async_copy(out_v, out_slice(t), osem).wait()

        def cast(bi):
            _, rows_v, out_v, _, _ = bufs[bi]

            def cast_body(gi, c2):
                for u in range(UNROLL):
                    j = gi * (UNROLL // 2) + u // 2
                    jv = jnp.full((L,), j, jnp.int32)
                    c0 = (u % 2) * 32
                    ev = plsc.load_gather(rows_v, [jv, c0 + 2 * iota])
                    od = plsc.load_gather(rows_v, [jv, c0 + 2 * iota + 1])
                    p = plsc.pack(ev, od, format=plsc.PackFormat.INTERLEAVED)
                    out_v[pl.ds(gi * (UNROLL * 32) + u * 32, 32)] = p
                return c2

            lax.fori_loop(0, CHUNK * D // (UNROLL * 32), cast_body, 0)

        start(0, 0)

        def body(p, carry):
            t0 = 2 * p
            t1 = 2 * p + 1
            start(t1, 1)
            wait_gathers(t0, 0)

            @pl.when(p > 0)
            def _():
                wait_out(t0 - 2, 0)

            cast(0)
            fire_out(t0, 0)

            @pl.when(p < nch // 2 - 1)
            def _():
                start(t0 + 2, 0)

            wait_gathers(t1, 1)

            @pl.when(p > 0)
            def _():
                wait_out(t1 - 2, 1)

            cast(1)
            fire_out(t1, 1)
            return carry

        lax.fori_loop(0, nch // 2, body, 0)
        wait_out(nch - 2, 0)
        wait_out(nch - 1, 1)

    return run(ids_flat, weight)


def kernel(input_ids, weight):
    b, s = input_ids.shape
    ids = input_ids.reshape(-1).astype(jnp.int32)
    out = _lookup(ids, weight)
    return out.reshape(b, s, D)


# speedup vs baseline: 11.9544x; 1.0024x over previous
import functools

import jax
import jax.numpy as jnp
from jax import lax
from jax.experimental import pallas as pl
from jax.experimental.pallas import tpu as pltpu
from jax.experimental.pallas import tpu_sc as plsc

D = 64
L = 16
CHUNK = 512
NG = 2
GROWS = CHUNK // NG
NW = 32
UNROLL = 16


def _lookup(ids_flat, weight):
    b_total = ids_flat.shape[0]
    per_w = b_total // NW
    nch = per_w // CHUNK
    assert nch % 2 == 0

    mesh = plsc.VectorSubcoreMesh(core_axis_name="c", subcore_axis_name="s")

    @functools.partial(
        pl.kernel,
        out_type=jax.ShapeDtypeStruct((b_total * D,), jnp.bfloat16),
        mesh=mesh,
        scratch_types=[
            pltpu.VMEM((CHUNK,), jnp.int32),
            pltpu.VMEM((CHUNK,), jnp.int32),
            pltpu.VMEM((CHUNK, D), jnp.float32),
            pltpu.VMEM((CHUNK, D), jnp.float32),
            pltpu.VMEM((CHUNK * D,), jnp.bfloat16),
            pltpu.VMEM((CHUNK * D,), jnp.bfloat16),
            pltpu.SemaphoreType.DMA,
            pltpu.SemaphoreType.DMA,
            pltpu.SemaphoreType.DMA,
            pltpu.SemaphoreType.DMA,
        ],
        compiler_params=pltpu.CompilerParams(
            needs_layout_passes=False, use_tc_tiling_on_sc=False
        ),
    )
    def run(idx_hbm, tbl_hbm, out_hbm, idx_a, idx_b, rows_a, rows_b,
            out_a, out_b, gsem_a, gsem_b, osem_a, osem_b):
        cid = lax.axis_index("c")
        sid = lax.axis_index("s")
        wid = sid * 2 + cid
        flat0 = wid * per_w
        oflat0 = wid * per_w * D

        iota = lax.iota(jnp.int32, L)
        bufs = ((idx_a, rows_a, out_a, gsem_a, osem_a),
                (idx_b, rows_b, out_b, gsem_b, osem_b))

        def start(t, bi):
            idx_v, rows_v, _, gsem, _ = bufs[bi]
            pltpu.sync_copy(
                idx_hbm.at[pl.ds(flat0 + t * CHUNK, CHUNK)], idx_v
            )
            for g in range(NG):
                pltpu.async_copy(
                    tbl_hbm.at[idx_v.at[pl.ds(g * GROWS, GROWS)]],
                    rows_v.at[pl.ds(g * GROWS, GROWS)],
                    gsem,
                )

        def wait_gathers(t, bi):
            idx_v, rows_v, _, gsem, _ = bufs[bi]
            for g in range(NG):
                pltpu.make_async_copy(
                    tbl_hbm.at[idx_v.at[pl.ds(g * GROWS, GROWS)]],
                    rows_v.at[pl.ds(g * GROWS, GROWS)],
                    gsem,
                ).wait()

        def out_slice(t):
            return out_hbm.at[pl.ds(oflat0 + t * CHUNK * D, CHUNK * D)]

        def fire_out(t, bi):
            _, _, out_v, _, osem = bufs[bi]
            pltpu.async_copy(out_v, out_slice(t), osem)

        def wait_out(t, bi):
            _, _, out_v, _, osem = bufs[bi]
            pltpu.make_async_copy(out_v, out_slice(t), osem).wait()

        def cast(bi):
            _, rows_v, out_v, _, _ = bufs[bi]

            def cast_body(gi, c2):
                for u in range(UNROLL):
                    j = gi * (UNROLL // 2) + u // 2
                    jv = jnp.full((L,), j, jnp.int32)
                    c0 = (u % 2) * 32
                    ev = plsc.load_gather(rows_v, [jv, c0 + 2 * iota])
                    od = plsc.load_gather(rows_v, [jv, c0 + 2 * iota + 1])
                    p = plsc.pack(ev, od, format=plsc.PackFormat.INTERLEAVED)
                    out_v[pl.ds(gi * (UNROLL * 32) + u * 32, 32)] = p
                return c2

            lax.fori_loop(0, CHUNK * D // (UNROLL * 32), cast_body, 0)

        start(0, 0)

        def body(p, carry):
            t0 = 2 * p
            t1 = 2 * p + 1
            start(t1, 1)
            wait_gathers(t0, 0)

            @pl.when(p > 0)
            def _():
                wait_out(t0 - 2, 0)

            cast(0)
            fire_out(t0, 0)

            @pl.when(p < nch // 2 - 1)
            def _():
                start(t0 + 2, 0)

            wait_gathers(t1, 1)

            @pl.when(p > 0)
            def _():
                wait_out(t1 - 2, 1)

            cast(1)
            fire_out(t1, 1)
            return carry

        lax.fori_loop(0, nch // 2, body, 0)
        wait_out(nch - 2, 0)
        wait_out(nch - 1, 1)

    return run(ids_flat, weight)


def kernel(input_ids, weight):
    b, s = input_ids.shape
    ids = input_ids.reshape(-1).astype(jnp.int32)
    out = _lookup(ids, weight)
    return out.reshape(b, s, D)
